# baseline (speedup 1.0000x reference)
import functools
import jax
import jax.numpy as jnp
from jax import lax
from jax.experimental import pallas as pl
from jax.experimental.pallas import tpu as pltpu
from jax.experimental.pallas import tpu_sc as plsc

_NUM_WORKERS = 32
_LANES = 16


@functools.partial(jax.jit, static_argnames=("seq_len", "num_visits"))
def _build_h(visit_boundaries, *, seq_len, num_visits):
    num_b = visit_boundaries.shape[0]
    rows_per_w = seq_len // _NUM_WORKERS
    groups = rows_per_w // _LANES
    mesh = plsc.VectorSubcoreMesh(core_axis_name="c", subcore_axis_name="s")

    @functools.partial(
        pl.kernel,
        out_type=jax.ShapeDtypeStruct((seq_len, num_visits), jnp.float32),
        mesh=mesh,
        scratch_types=[
            pltpu.VMEM((num_b,), jnp.int32),
            pltpu.VMEM((rows_per_w, num_visits), jnp.float32),
            pltpu.SemaphoreType.DMA,
        ],
        compiler_params=pltpu.CompilerParams(needs_layout_passes=False),
    )
    def k(vb_hbm, out_hbm, vb, buf, sem):
        wid = lax.axis_index("s") * 2 + lax.axis_index("c")
        vb_copy = pltpu.async_copy(vb_hbm, vb, sem)

        lane = lax.iota(jnp.int32, _LANES)
        zeros = jnp.zeros((_LANES,), jnp.float32)
        n_full = num_visits // _LANES
        col_starts = [c * _LANES for c in range(n_full)]
        if num_visits % _LANES:
            col_starts.append(num_visits - _LANES)

        row_unroll = 4

        def zero_body(i, carry):
            r0 = i * row_unroll
            for u in range(row_unroll):
                row = jnp.full((_LANES,), u, jnp.int32) + r0
                for c in col_starts:
                    plsc.store_scatter(buf, [row, c + lane], zeros)
            return carry

        base = wid * rows_per_w
        ones = jnp.ones((_LANES,), jnp.float32)
        copies = []
        for g in range(groups):
            lax.fori_loop(
                g * (_LANES // row_unroll),
                (g + 1) * (_LANES // row_unroll),
                zero_body,
                0,
            )
            if g == 0:
                vb_copy.wait()
            local = g * _LANES + lane
            r = base + local
            pos = jnp.zeros((_LANES,), jnp.int32)
            s = num_b // 2
            while s >= 1:
                val = plsc.load_gather(vb, [pos + (s - 1)])
                pos = jnp.where(val <= r, pos + s, pos)
                s //= 2
            val = plsc.load_gather(vb, [pos])
            pos = jnp.where(val <= r, pos + 1, pos)
            plsc.store_scatter(buf, [local, pos], ones)
            copies.append(
                pltpu.async_copy(
                    buf.at[pl.ds(g * _LANES, _LANES)],
                    out_hbm.at[pl.ds(wid * rows_per_w + g * _LANES, _LANES)],
                    sem,
                )
            )
        for c in copies:
            c.wait()

    return k(visit_boundaries)


def kernel(X, visit_boundaries):
    seq_len = X.shape[0]
    num_visits = visit_boundaries.shape[0] + 1
    return _build_h(
        visit_boundaries.astype(jnp.int32), seq_len=seq_len, num_visits=num_visits
    ).astype(X.dtype)


# speedup vs baseline: 1.0869x; 1.0869x over previous
import functools
import jax
import jax.numpy as jnp
from jax import lax
from jax.experimental import pallas as pl
from jax.experimental.pallas import tpu as pltpu
from jax.experimental.pallas import tpu_sc as plsc

_NUM_WORKERS = 32
_LANES = 16


@functools.partial(jax.jit, static_argnames=("seq_len", "num_visits"))
def _build_h(visit_boundaries, *, seq_len, num_visits):
    num_b = visit_boundaries.shape[0]
    rows_per_w = seq_len // _NUM_WORKERS
    groups = rows_per_w // _LANES
    mesh = plsc.VectorSubcoreMesh(core_axis_name="c", subcore_axis_name="s")

    @functools.partial(
        pl.kernel,
        out_type=jax.ShapeDtypeStruct((seq_len, num_visits), jnp.float32),
        mesh=mesh,
        scratch_types=[
            pltpu.VMEM((num_b,), jnp.int32),
            pltpu.VMEM((rows_per_w, num_visits), jnp.float32),
            pltpu.SemaphoreType.DMA,
        ],
        compiler_params=pltpu.CompilerParams(needs_layout_passes=False),
    )
    def k(vb_hbm, out_hbm, vb, buf, sem):
        wid = lax.axis_index("s") * 2 + lax.axis_index("c")
        vb_copy = pltpu.async_copy(vb_hbm, vb, sem)

        lane = lax.iota(jnp.int32, _LANES)
        zeros = jnp.zeros((_LANES,), jnp.float32)
        n_full = num_visits // _LANES
        col_starts = [c * _LANES for c in range(n_full)]
        if num_visits % _LANES:
            col_starts.append(num_visits - _LANES)

        row_unroll = 4

        def zero_body(i, carry):
            r0 = i * row_unroll
            for u in range(row_unroll):
                row = jnp.full((_LANES,), u, jnp.int32) + r0
                for c in col_starts:
                    plsc.store_scatter(buf, [row, c + lane], zeros)
            return carry

        lax.fori_loop(0, rows_per_w // row_unroll, zero_body, 0)
        vb_copy.wait()

        base = wid * rows_per_w
        ones = jnp.ones((_LANES,), jnp.float32)
        for g in range(groups):
            local = g * _LANES + lane
            r = base + local
            pos = jnp.zeros((_LANES,), jnp.int32)
            s = num_b // 2
            while s >= 1:
                val = plsc.load_gather(vb, [pos + (s - 1)])
                pos = jnp.where(val <= r, pos + s, pos)
                s //= 2
            val = plsc.load_gather(vb, [pos])
            pos = jnp.where(val <= r, pos + 1, pos)
            plsc.store_scatter(buf, [local, pos], ones)

        pltpu.sync_copy(buf, out_hbm.at[pl.ds(wid * rows_per_w, rows_per_w)])

    return k(visit_boundaries)


def kernel(X, visit_boundaries):
    seq_len = X.shape[0]
    num_visits = visit_boundaries.shape[0] + 1
    return _build_h(
        visit_boundaries.astype(jnp.int32), seq_len=seq_len, num_visits=num_visits
    ).astype(X.dtype)


# speedup vs baseline: 1.1185x; 1.0291x over previous
import functools
import jax
import jax.numpy as jnp
from jax import lax
from jax.experimental import pallas as pl
from jax.experimental.pallas import tpu as pltpu
from jax.experimental.pallas import tpu_sc as plsc

_NUM_WORKERS = 32
_LANES = 16


@functools.partial(jax.jit, static_argnames=("seq_len", "num_visits"))
def _build_h(visit_boundaries, *, seq_len, num_visits):
    num_b = visit_boundaries.shape[0]
    rows_per_w = seq_len // _NUM_WORKERS
    groups = rows_per_w // _LANES
    mesh = plsc.VectorSubcoreMesh(core_axis_name="c", subcore_axis_name="s")

    @functools.partial(
        pl.kernel,
        out_type=jax.ShapeDtypeStruct((seq_len, num_visits), jnp.float32),
        mesh=mesh,
        scratch_types=[
            pltpu.VMEM((num_b,), jnp.int32),
            pltpu.VMEM((rows_per_w, num_visits), jnp.float32),
            pltpu.SemaphoreType.DMA,
        ],
        compiler_params=pltpu.CompilerParams(needs_layout_passes=False),
    )
    def k(vb_hbm, out_hbm, vb, buf, sem):
        wid = lax.axis_index("s") * 2 + lax.axis_index("c")
        vb_copy = pltpu.async_copy(vb_hbm, vb, sem)

        lane = lax.iota(jnp.int32, _LANES)
        zeros = jnp.zeros((_LANES,), jnp.float32)
        n_full = num_visits // _LANES
        col_starts = [c * _LANES for c in range(n_full)]
        if num_visits % _LANES:
            col_starts.append(num_visits - _LANES)

        row_unroll = 4

        def zero_body(i, carry):
            r0 = i * row_unroll
            for u in range(row_unroll):
                row = jnp.full((_LANES,), u, jnp.int32) + r0
                for c in col_starts:
                    plsc.store_scatter(buf, [row, c + lane], zeros)
            return carry

        lax.fori_loop(0, rows_per_w // row_unroll, zero_body, 0)
        vb_copy.wait()

        base = wid * rows_per_w
        ones = jnp.ones((_LANES,), jnp.float32)
        def group_body(g, carry):
            local = g * _LANES + lane
            r = base + local
            pos = jnp.zeros((_LANES,), jnp.int32)
            s = num_b // 2
            while s >= 1:
                val = plsc.load_gather(vb, [pos + (s - 1)])
                pos = jnp.where(val <= r, pos + s, pos)
                s //= 2
            val = plsc.load_gather(vb, [pos])
            pos = jnp.where(val <= r, pos + 1, pos)
            plsc.store_scatter(buf, [local, pos], ones)
            return carry

        lax.fori_loop(0, groups, group_body, 0)

        pltpu.sync_copy(buf, out_hbm.at[pl.ds(wid * rows_per_w, rows_per_w)])

    return k(visit_boundaries)


def kernel(X, visit_boundaries):
    seq_len = X.shape[0]
    num_visits = visit_boundaries.shape[0] + 1
    return _build_h(
        visit_boundaries.astype(jnp.int32), seq_len=seq_len, num_visits=num_visits
    ).astype(X.dtype)
